# baseline (speedup 1.0000x reference)
import numpy as np

import jax
import jax.numpy as jnp
from jax import lax
from jax.experimental import pallas as pl
from jax.experimental.pallas import tpu as pltpu
from jax.experimental.pallas import tpu_sc as plsc

NUM_FACTORS = 3
VOCAB_P1 = 513
D = 2048
B = 4
S = 8192
N = B * S

NC = 2
NS = 16
LANES = 16
NW = NC * NS
NT = N // NW
T = 8
NCHUNK = NT // T
GROUPS_PER_ROW = D // (2 * LANES)
HIMASK = -65536

_SRC = np.empty((D,), dtype=np.int32)
for _g in range(GROUPS_PER_ROW):
  for _j in range(LANES):
    _SRC[32 * _g + 2 * _j] = 32 * _g + _j
    _SRC[32 * _g + 2 * _j + 1] = 32 * _g + LANES + _j


def _body(w0, w1, w2, i0, i1, i2, out,
          idx0_v, idx1_v, idx2_v,
          ob0, ob1, ob2, g1b0, g1b1, g1b2, g2b0, g2b1, g2b2,
          s00, s01, s02, s10, s11, s12, s20, s21, s22, st0, st1, st2):
  wid = lax.axis_index("s") * NC + lax.axis_index("c")
  base = wid * NT

  obufs = (ob0, ob1, ob2)
  g1bufs = (g1b0, g1b1, g1b2)
  g2bufs = (g2b0, g2b1, g2b2)
  sems = ((s00, s10, s20), (s01, s11, s21), (s02, s12, s22))
  stsems = (st0, st1, st2)

  pltpu.sync_copy(i0.at[wid], idx0_v)
  pltpu.sync_copy(i1.at[wid], idx1_v)
  pltpu.sync_copy(i2.at[wid], idx2_v)

  def issue(c, s):
    pltpu.async_copy(w0.at[idx0_v.at[pl.ds(c * T, T)]], obufs[s], sems[s][0])
    pltpu.async_copy(w1.at[idx1_v.at[pl.ds(c * T, T)]], g1bufs[s], sems[s][1])
    pltpu.async_copy(w2.at[idx2_v.at[pl.ds(c * T, T)]], g2bufs[s], sems[s][2])

  def drain(c, s):
    pltpu.make_async_copy(w0.at[idx0_v.at[pl.ds(c * T, T)]], obufs[s],
                          sems[s][0]).wait()
    pltpu.make_async_copy(w1.at[idx1_v.at[pl.ds(c * T, T)]], g1bufs[s],
                          sems[s][1]).wait()
    pltpu.make_async_copy(w2.at[idx2_v.at[pl.ds(c * T, T)]], g2bufs[s],
                          sems[s][2]).wait()

  def fold_store(c, s):
    ob, g1, g2 = obufs[s], g1bufs[s], g2bufs[s]

    def row_body(r, rcarry):
      for v in range(GROUPS_PER_ROW):
        colw = v * LANES
        col = v * 2 * LANES
        x1 = g1[r, pl.ds(colw, LANES)]
        x2 = g2[r, pl.ds(colw, LANES)]
        a = (lax.bitcast_convert_type(x1 << 16, jnp.float32)
             + lax.bitcast_convert_type(x2 << 16, jnp.float32))
        b = (lax.bitcast_convert_type(x1, jnp.float32)
             + lax.bitcast_convert_type(x2, jnp.float32))
        plsc.addupdate(ob.at[r, pl.ds(col, LANES)], a)
        plsc.addupdate(ob.at[r, pl.ds(col + LANES, LANES)], b)
      return rcarry

    lax.fori_loop(0, T, row_body, 0, unroll=False)
    pltpu.async_copy(ob, out.at[pl.ds(base + c * T, T)], stsems[s])

  def drain_store(s):
    pltpu.make_async_copy(obufs[s], out.at[pl.ds(base, T)], stsems[s]).wait()

  def phase(c, s, sp, first):
    drain(c, s)
    if first:
      @pl.when(c > 0)
      def _():
        drain_store(sp)
    else:
      drain_store(sp)
    issue(jnp.minimum(c + 2, NCHUNK - 1), sp)
    fold_store(c, s)

  issue(0, 0)
  issue(1, 1)

  NTRIPLE = NCHUNK // 3

  def triple_body(q, carry):
    c = 3 * q
    phase(c, 0, 2, True)
    phase(c + 1, 1, 0, False)
    phase(c + 2, 2, 1, False)
    return carry

  lax.fori_loop(0, NTRIPLE, triple_body, 0, unroll=False)
  for c in range(3 * NTRIPLE, NCHUNK):
    phase(c, c % 3, (c + 2) % 3, False)
  drain_store((NCHUNK - 1) % 3)
  drain(NCHUNK - 1, (NCHUNK + 1) % 3)
  drain(NCHUNK - 1, NCHUNK % 3)


@jax.jit
def kernel(x, W0, W1, W2):
  src = jnp.asarray(_SRC)

  def prep(w):
    wb = w[:, src].astype(jnp.bfloat16).reshape(VOCAB_P1, D // 2, 2)
    return lax.bitcast_convert_type(wb, jnp.int32)

  wb1, wb2 = prep(W1), prep(W2)
  xt = jnp.transpose(x.astype(jnp.int32), (1, 0, 2)).reshape(
      NUM_FACTORS, NW, NT)
  mesh = plsc.VectorSubcoreMesh(core_axis_name="c", subcore_axis_name="s",
                                num_cores=NC, num_subcores=NS)
  fn = pl.kernel(
      _body,
      out_type=jax.ShapeDtypeStruct((N, D), jnp.float32),
      mesh=mesh,
      scratch_types=[
          pltpu.VMEM((NT,), jnp.int32),
          pltpu.VMEM((NT,), jnp.int32),
          pltpu.VMEM((NT,), jnp.int32),
          pltpu.VMEM((T, D), jnp.float32),
          pltpu.VMEM((T, D), jnp.float32),
          pltpu.VMEM((T, D), jnp.float32),
          pltpu.VMEM((T, D // 2), jnp.int32),
          pltpu.VMEM((T, D // 2), jnp.int32),
          pltpu.VMEM((T, D // 2), jnp.int32),
          pltpu.VMEM((T, D // 2), jnp.int32),
          pltpu.VMEM((T, D // 2), jnp.int32),
          pltpu.VMEM((T, D // 2), jnp.int32),
          pltpu.SemaphoreType.DMA,
          pltpu.SemaphoreType.DMA,
          pltpu.SemaphoreType.DMA,
          pltpu.SemaphoreType.DMA,
          pltpu.SemaphoreType.DMA,
          pltpu.SemaphoreType.DMA,
          pltpu.SemaphoreType.DMA,
          pltpu.SemaphoreType.DMA,
          pltpu.SemaphoreType.DMA,
          pltpu.SemaphoreType.DMA,
          pltpu.SemaphoreType.DMA,
          pltpu.SemaphoreType.DMA,
      ],
  )
  out = fn(W0, wb1, wb2, xt[0], xt[1], xt[2])
  return out.reshape(B, S, D)
